# baseline (speedup 1.0000x reference)
import dataclasses
import functools

import jax
import jax.numpy as jnp
from jax import lax
from jax.experimental import pallas as pl
from jax.experimental.pallas import tpu as pltpu
from jax.experimental.pallas import tpu_sc as plsc

N_NODES = 10000
N_EDGES = 320000
D = 128

N_TILES = 32
EDGES_PAD = 327680
EDGES_PER_TILE = EDGES_PAD // N_TILES
CHUNK = 256
N_CHUNKS = EDGES_PER_TILE // CHUNK
IDX_ROWS = CHUNK // 128


def _normalize_body(x_ref, o_ref):
    xb = x_ref[...]
    ss = jnp.sum(xb * xb, axis=1, keepdims=True)
    norm = jnp.maximum(jnp.sqrt(ss), 1e-12)
    o_ref[...] = xb / norm


def _normalize(x):
    return pl.pallas_call(
        _normalize_body,
        out_shape=jax.ShapeDtypeStruct((N_NODES, D), jnp.float32),
        grid=(10,),
        in_specs=[pl.BlockSpec((N_NODES // 10, D), lambda i: (i, 0))],
        out_specs=pl.BlockSpec((N_NODES // 10, D), lambda i: (i, 0)),
    )(x)


def _sc_cosine(norm_h, src2d, dst2d):
    mesh = plsc.VectorSubcoreMesh(core_axis_name="c", subcore_axis_name="s")
    cp = pltpu.CompilerParams()
    if "needs_layout_passes" in pltpu.CompilerParams.__dataclass_fields__:
        cp = dataclasses.replace(cp, needs_layout_passes=False)

    @functools.partial(
        pl.kernel,
        mesh=mesh,
        compiler_params=cp,
        out_type=jax.ShapeDtypeStruct((EDGES_PAD,), jnp.float32),
        scratch_types=[
            pltpu.VMEM((IDX_ROWS, 128), jnp.int32),
            pltpu.VMEM((IDX_ROWS, 128), jnp.int32),
            pltpu.VMEM((CHUNK, D), jnp.float32),
            pltpu.VMEM((CHUNK, D), jnp.float32),
            pltpu.VMEM((CHUNK,), jnp.float32),
            pltpu.SemaphoreType.DMA,
        ],
    )
    def sc_kernel(h_hbm, src_hbm, dst_hbm, out_hbm,
                  sidx, didx, urows, vrows, res, sem):
        wid = lax.axis_index("s") * 2 + lax.axis_index("c")
        row0 = wid * (EDGES_PER_TILE // 128)

        @pl.loop(0, N_CHUNKS)
        def _chunk(c):
            r = row0 + c * IDX_ROWS
            pltpu.sync_copy(src_hbm.at[pl.ds(r, IDX_ROWS)], sidx)
            pltpu.sync_copy(dst_hbm.at[pl.ds(r, IDX_ROWS)], didx)
            copies = []
            for k in range(IDX_ROWS):
                copies.append(pltpu.async_copy(
                    h_hbm.at[sidx.at[k]],
                    urows.at[pl.ds(k * 128, 128)], sem))
                copies.append(pltpu.async_copy(
                    h_hbm.at[didx.at[k]],
                    vrows.at[pl.ds(k * 128, 128)], sem))
            for cp in copies:
                cp.wait()

            @pl.loop(0, CHUNK // 16)
            def _group(g):
                rows = g * 16 + lax.iota(jnp.int32, 16)

                def jstep(jj, acc):
                    j0 = jj * 8
                    for k in range(8):
                        jv = jnp.broadcast_to(j0 + k, (16,)).astype(jnp.int32)
                        uu = plsc.load_gather(urows, [rows, jv])
                        vv = plsc.load_gather(vrows, [rows, jv])
                        acc = acc + uu * vv
                    return acc

                acc = lax.fori_loop(0, D // 8, jstep,
                                    jnp.zeros((16,), jnp.float32))
                res[pl.ds(g * 16, 16)] = acc

            pltpu.sync_copy(res, out_hbm.at[pl.ds(wid * EDGES_PER_TILE
                                                  + c * CHUNK, CHUNK)])

    return sc_kernel(norm_h, src2d, dst2d)


def kernel(x, edge_index):
    norm_h = _normalize(x.astype(jnp.float32))
    ei = edge_index.astype(jnp.int32)
    pad = EDGES_PAD - N_EDGES
    src = jnp.concatenate([ei[0], jnp.zeros((pad,), jnp.int32)])
    dst = jnp.concatenate([ei[1], jnp.zeros((pad,), jnp.int32)])
    src2d = src.reshape(EDGES_PAD // 128, 128)
    dst2d = dst.reshape(EDGES_PAD // 128, 128)
    cos = _sc_cosine(norm_h, src2d, dst2d)
    return cos[:N_EDGES].reshape(N_EDGES, 1)
